# baseline (speedup 1.0000x reference)
import functools
import jax
import jax.numpy as jnp
from jax import lax
from jax.experimental import pallas as pl
from jax.experimental.pallas import tpu as pltpu
from jax.experimental.pallas import tpu_sc as plsc

HIDDEN = 64
NC = 2
NS = 16
NW = NC * NS
CHUNK = 1024


def _make_gather(total, hidden):
    b_per_w = total // NW
    nchunk = b_per_w // CHUNK
    mesh = plsc.VectorSubcoreMesh(core_axis_name="c", subcore_axis_name="s")

    @functools.partial(
        pl.kernel,
        mesh=mesh,
        out_type=jax.ShapeDtypeStruct((total, hidden), jnp.float32),
        scratch_types=[
            pltpu.VMEM((CHUNK,), jnp.int32),
            pltpu.VMEM((CHUNK, hidden), jnp.float32),
            pltpu.SemaphoreType.DMA,
        ],
        compiler_params=pltpu.CompilerParams(use_tc_tiling_on_sc=False),
    )
    def gather_kernel(idx_hbm, table_hbm, out_hbm, idx_v, rows_v, sem):
        wid = lax.axis_index("s") * NC + lax.axis_index("c")
        base = wid * b_per_w

        def body(i, carry):
            off = base + i * CHUNK
            pltpu.sync_copy(idx_hbm.at[pl.ds(off, CHUNK)], idx_v)
            pltpu.async_copy(table_hbm.at[idx_v], rows_v, sem).wait()
            pltpu.sync_copy(rows_v, out_hbm.at[pl.ds(off, CHUNK)])
            return carry

        lax.fori_loop(0, nchunk, body, 0)

    return gather_kernel


def kernel(input_ids, weight):
    batch, seq = input_ids.shape
    vocab, hidden = weight.shape
    total = batch * seq
    flat_ids = input_ids.reshape(total).astype(jnp.int32)
    out = _make_gather(total, hidden)(flat_ids, weight)
    return out.reshape(batch, seq, hidden)


# speedup vs baseline: 1.0131x; 1.0131x over previous
import functools
import jax
import jax.numpy as jnp
from jax import lax
from jax.experimental import pallas as pl
from jax.experimental.pallas import tpu as pltpu
from jax.experimental.pallas import tpu_sc as plsc

HIDDEN = 64
NC = 2
NS = 16
NW = NC * NS
CHUNK = 512
NBUF = 3


def _make_gather(total, hidden):
    b_per_w = total // NW
    nchunk = b_per_w // CHUNK
    assert nchunk % NBUF == 2, "pipeline peels slot 0 and the last slot"
    mesh = plsc.VectorSubcoreMesh(core_axis_name="c", subcore_axis_name="s")

    @functools.partial(
        pl.kernel,
        mesh=mesh,
        out_type=jax.ShapeDtypeStruct((total, hidden), jnp.float32),
        scratch_types=[
            pltpu.VMEM((b_per_w,), jnp.int32),
            pltpu.VMEM((NBUF, CHUNK, hidden), jnp.float32),
            pltpu.SemaphoreType.DMA,
            pltpu.SemaphoreType.DMA,
            pltpu.SemaphoreType.DMA,
            pltpu.SemaphoreType.DMA,
            pltpu.SemaphoreType.DMA,
            pltpu.SemaphoreType.DMA,
        ],
        compiler_params=pltpu.CompilerParams(use_tc_tiling_on_sc=False),
    )
    def gather_kernel(idx_hbm, table_hbm, out_hbm, idx_v, rows_v,
                      sg0, sg1, sg2, ss0, ss1, ss2):
        wid = lax.axis_index("s") * NC + lax.axis_index("c")
        base = wid * b_per_w
        pltpu.sync_copy(idx_hbm.at[pl.ds(base, b_per_w)], idx_v)

        sg = (sg0, sg1, sg2)
        ss = (ss0, ss1, ss2)
        rows = tuple(rows_v.at[b] for b in range(NBUF))

        def start_gather(chunk, b):
            pltpu.async_copy(
                table_hbm.at[idx_v.at[pl.ds(chunk * CHUNK, CHUNK)]],
                rows[b], sg[b])

        def wait_gather(b):
            pltpu.make_async_copy(
                table_hbm.at[idx_v.at[pl.ds(0, CHUNK)]],
                rows[b], sg[b]).wait()

        def start_scatter(slot, b):
            pltpu.async_copy(
                rows[b], out_hbm.at[pl.ds(base + slot * CHUNK, CHUNK)], ss[b])

        def wait_scatter(b):
            pltpu.make_async_copy(
                rows[b], out_hbm.at[pl.ds(0, CHUNK)], ss[b]).wait()

        start_gather(0, 0)
        start_gather(1, 1)
        wait_gather(0)
        start_scatter(0, 0)
        start_gather(2, 2)

        def body(j, carry):
            for t in range(NBUF):
                slot = 3 * j + 1 + t
                b = (1 + t) % NBUF
                c = t
                wait_gather(b)
                start_scatter(slot, b)
                wait_scatter(c)

                @pl.when(slot + 2 < nchunk)
                def _():
                    start_gather(slot + 2, c)
            return carry

        lax.fori_loop(0, (nchunk - 2) // NBUF, body, 0)

        last = nchunk - 1
        bl = last % NBUF
        wait_gather(bl)
        start_scatter(last, bl)
        wait_scatter((last + 2) % NBUF)
        wait_scatter(bl)

    return gather_kernel


def kernel(input_ids, weight):
    batch, seq = input_ids.shape
    vocab, hidden = weight.shape
    total = batch * seq
    flat_ids = input_ids.reshape(total).astype(jnp.int32)
    out = _make_gather(total, hidden)(flat_ids, weight)
    return out.reshape(batch, seq, hidden)


# speedup vs baseline: 1.0712x; 1.0573x over previous
import functools
import jax
import jax.numpy as jnp
from jax import lax
from jax.experimental import pallas as pl
from jax.experimental.pallas import tpu as pltpu
from jax.experimental.pallas import tpu_sc as plsc

HIDDEN = 64
NC = 2
NS = 16
NW = NC * NS
CHUNK = 512
NBUF = 3


def _make_gather(total, hidden):
    b_per_w = total // NW
    nchunk = b_per_w // CHUNK
    assert nchunk % NBUF == 2, "pipeline peels slot 0 and the last slot"
    mesh = plsc.VectorSubcoreMesh(core_axis_name="c", subcore_axis_name="s")

    @functools.partial(
        pl.kernel,
        mesh=mesh,
        out_type=jax.ShapeDtypeStruct((total, hidden), jnp.float32),
        scratch_types=[
            pltpu.VMEM((b_per_w,), jnp.int32),
            pltpu.VMEM((NBUF, CHUNK, hidden), jnp.float32),
            pltpu.SemaphoreType.DMA,
            pltpu.SemaphoreType.DMA,
            pltpu.SemaphoreType.DMA,
            pltpu.SemaphoreType.DMA,
            pltpu.SemaphoreType.DMA,
            pltpu.SemaphoreType.DMA,
        ],
        compiler_params=pltpu.CompilerParams(use_tc_tiling_on_sc=False),
    )
    def gather_kernel(idx_hbm, table_hbm, out_hbm, idx_v, rows_v,
                      sg0, sg1, sg2, ss0, ss1, ss2):
        wid = lax.axis_index("s") * NC + lax.axis_index("c")
        base = wid * b_per_w
        pltpu.sync_copy(idx_hbm.at[pl.ds(base, b_per_w)], idx_v)

        sg = (sg0, sg1, sg2)
        ss = (ss0, ss1, ss2)
        rows = tuple(rows_v.at[b] for b in range(NBUF))

        def start_gather(chunk, b):
            pltpu.async_copy(
                table_hbm.at[idx_v.at[pl.ds(chunk * CHUNK, CHUNK)]],
                rows[b], sg[b])

        def wait_gather(b):
            pltpu.make_async_copy(
                table_hbm.at[idx_v.at[pl.ds(0, CHUNK)]],
                rows[b], sg[b]).wait()

        def start_scatter(slot, b):
            pltpu.async_copy(
                rows[b], out_hbm.at[pl.ds(base + slot * CHUNK, CHUNK)], ss[b])

        def wait_scatter(b):
            pltpu.make_async_copy(
                rows[b], out_hbm.at[pl.ds(0, CHUNK)], ss[b]).wait()

        start_gather(0, 0)
        start_gather(1, 1)
        wait_gather(0)
        start_scatter(0, 0)
        start_gather(2, 2)

        def body(j, carry):
            for t in range(NBUF):
                slot = 3 * j + 1 + t
                b = (1 + t) % NBUF
                c = t
                wait_gather(b)
                start_scatter(slot, b)
                wait_scatter(c)

                @pl.when(slot + 2 < nchunk)
                def _():
                    start_gather(slot + 2, c)
            return carry

        lax.fori_loop(0, (nchunk - 2) // NBUF, body, 0)

        last = nchunk - 1
        bl = last % NBUF
        wait_gather(bl)
        start_scatter(last, bl)
        wait_scatter((last + 2) % NBUF)
        wait_scatter(bl)

    return gather_kernel


def kernel(input_ids, weight):
    batch, seq = input_ids.shape
    vocab, hidden = weight.shape
    total = batch * seq
    w_pad = jnp.pad(weight, ((0, 0), (0, 128 - hidden))).reshape(2 * vocab, hidden)
    flat_ids = (input_ids.reshape(total) * 2).astype(jnp.int32)
    out = _make_gather(total, hidden)(flat_ids, w_pad)
    return out.reshape(batch, seq, hidden)


# speedup vs baseline: 1.1246x; 1.0499x over previous
import functools
import jax
import jax.numpy as jnp
from jax import lax
from jax.experimental import pallas as pl
from jax.experimental.pallas import tpu as pltpu
from jax.experimental.pallas import tpu_sc as plsc

HIDDEN = 64
NC = 2
NS = 16
NW = NC * NS
CHUNK = 512
NBUF = 3


def _make_gather(total, hidden):
    b_per_w = total // NW
    nchunk = b_per_w // CHUNK
    assert nchunk % NBUF == 2, "pipeline peels slot 0 and the last slot"
    mesh = plsc.VectorSubcoreMesh(core_axis_name="c", subcore_axis_name="s")

    @functools.partial(
        pl.kernel,
        mesh=mesh,
        out_type=jax.ShapeDtypeStruct((total, hidden), jnp.float32),
        scratch_types=[
            pltpu.VMEM((b_per_w,), jnp.int32),
            pltpu.VMEM((NBUF, CHUNK, hidden), jnp.float32),
            pltpu.SemaphoreType.DMA,
            pltpu.SemaphoreType.DMA,
            pltpu.SemaphoreType.DMA,
            pltpu.SemaphoreType.DMA,
            pltpu.SemaphoreType.DMA,
            pltpu.SemaphoreType.DMA,
        ],
        compiler_params=pltpu.CompilerParams(use_tc_tiling_on_sc=False),
    )
    def gather_kernel(idx_hbm, table_hbm, out_hbm, idx_v, rows_v,
                      sg0, sg1, sg2, ss0, ss1, ss2):
        wid = lax.axis_index("s") * NC + lax.axis_index("c")
        base = wid * b_per_w
        pltpu.sync_copy(idx_hbm.at[pl.ds(base, b_per_w)], idx_v)

        sg = (sg0, sg1, sg2)
        ss = (ss0, ss1, ss2)
        rows = tuple(rows_v.at[b] for b in range(NBUF))

        def start_gather(chunk, b):
            pltpu.async_copy(
                table_hbm.at[idx_v.at[pl.ds(chunk * CHUNK, CHUNK)]],
                rows[b], sg[b])

        def wait_gather(b):
            pltpu.make_async_copy(
                table_hbm.at[idx_v.at[pl.ds(0, CHUNK)]],
                rows[b], sg[b]).wait()

        def start_scatter(slot, b):
            pltpu.async_copy(
                rows[b], out_hbm.at[pl.ds(base + slot * CHUNK, CHUNK)], ss[b])

        def wait_scatter(b):
            pltpu.make_async_copy(
                rows[b], out_hbm.at[pl.ds(0, CHUNK)], ss[b]).wait()

        start_gather(0, 0)
        start_gather(1, 1)
        wait_gather(0)
        start_scatter(0, 0)
        start_gather(2, 2)

        def body(j, carry):
            for t in range(NBUF):
                slot = 3 * j + 1 + t
                b = (1 + t) % NBUF
                c = t
                wait_gather(b)
                start_scatter(slot, b)
                wait_scatter(c)

                @pl.when(slot + 2 < nchunk)
                def _():
                    start_gather(slot + 2, c)
            return carry

        lax.fori_loop(0, (nchunk - 2) // NBUF, body, 0)

        last = nchunk - 1
        bl = last % NBUF
        wait_gather(bl)
        start_scatter(last, bl)
        wait_scatter((last + 2) % NBUF)
        wait_scatter(bl)

    return gather_kernel


def _transpose_pad(weight):
    vocab, hidden = weight.shape
    wt = weight.T
    vb = 2048
    grid = (vocab + vb - 1) // vb

    def body(wt_ref, out_ref):
        out_ref[:, 0:hidden] = wt_ref[...].T
        out_ref[:, hidden:128] = jnp.zeros((vb, 128 - hidden), jnp.float32)

    return pl.pallas_call(
        body,
        grid=(grid,),
        in_specs=[pl.BlockSpec((hidden, vb), lambda j: (0, j))],
        out_specs=pl.BlockSpec((vb, 128), lambda j: (j, 0)),
        out_shape=jax.ShapeDtypeStruct((vocab, 128), jnp.float32),
    )(wt)


def kernel(input_ids, weight):
    batch, seq = input_ids.shape
    vocab, hidden = weight.shape
    total = batch * seq
    w_pad = _transpose_pad(weight).reshape(2 * vocab, hidden)
    flat_ids = (input_ids.reshape(total) * 2).astype(jnp.int32)
    out = _make_gather(total, hidden)(flat_ids, w_pad)
    return out.reshape(batch, seq, hidden)
